# baseline (speedup 1.0000x reference)
import functools

import jax
import jax.numpy as jnp
from jax import lax
from jax.experimental import pallas as pl
from jax.experimental.pallas import tpu as pltpu
from jax.experimental.pallas import tpu_sc as plsc

N = 10000
E = 320000
G = 128
NC = 2
NS = 16
HN = N // NC
CH = 125
EPT = E // NS
NCH = EPT // CH
NBUF = 2
SPAN, CZ = 320, 40

_F32 = jnp.float32


@functools.cache
def _mesh():
  return plsc.VectorSubcoreMesh(core_axis_name="c", subcore_axis_name="s")


@functools.cache
def _agg_sc():
  return pl.kernel(
      _agg_sc_body,
      out_type=jax.ShapeDtypeStruct((NC, HN, 128), _F32),
      mesh=_mesh(),
      scratch_types=[
          pltpu.VMEM((NCH, CH), jnp.int32),
          pltpu.VMEM((NCH, CH), jnp.int32),
          [pltpu.VMEM((CH, 128), _F32)] * NBUF,
          pltpu.VMEM((CZ, 128), _F32),
          pltpu.VMEM_SHARED((HN + 8, 128), _F32),
          [pltpu.SemaphoreType.DMA] * NBUF,
      ])


def _agg_sc_body(t_hbm, src_hbm, dst0_hbm, dst1_hbm, zeros_hbm, out_hbm,
                 idx_s, idx_d, rows, zbuf, acc, sem):
  c = lax.axis_index("c")
  s = lax.axis_index("s")
  pltpu.sync_copy(src_hbm.at[s], idx_s)

  @pl.when(c == 0)
  def _():
    pltpu.sync_copy(dst0_hbm.at[s], idx_d)

  @pl.when(c == 1)
  def _():
    pltpu.sync_copy(dst1_hbm.at[s], idx_d)

  pltpu.sync_copy(zeros_hbm.at[pl.ds(0, CZ)], zbuf)

  def zcp(k, _):
    pltpu.sync_copy(zbuf, acc.at[pl.ds(s * SPAN + k * CZ, CZ)])
    return 0

  lax.fori_loop(0, 5, zcp, 0)

  @pl.when(s < NS - 1)
  def _():
    lax.fori_loop(5, 8, zcp, 0)

  plsc.subcore_barrier()

  def group(g, _):
    base = NBUF * g
    started = [
        pltpu.async_copy(t_hbm.at[idx_s.at[base + k]], rows[k], sem[k])
        for k in range(NBUF)
    ]
    for k in range(NBUF):
      started[k].wait()
      pltpu.sync_copy(rows[k], acc.at[idx_d.at[base + k]], add=True)
    return 0

  lax.fori_loop(0, NCH // NBUF, group, 0)
  plsc.subcore_barrier()

  def rcp(k, _):
    o = s * SPAN + k * CZ
    pltpu.sync_copy(acc.at[pl.ds(o, CZ)], out_hbm.at[c, pl.ds(o, CZ)])
    return 0

  lax.fori_loop(0, 5, rcp, 0)

  @pl.when(s < NS - 1)
  def _():
    lax.fori_loop(5, 8, rcp, 0)


@functools.cache
def _pool_sc():
  return pl.kernel(
      _pool_sc_body,
      out_type=jax.ShapeDtypeStruct((4, 8, G, 128), _F32),
      mesh=_mesh(),
      scratch_types=[
          pltpu.VMEM((632, 128), _F32),
          pltpu.VMEM((1280,), jnp.int32),
          pltpu.VMEM((G, 128), _F32),
      ])


def _pool_sc_body(h3_hbm, batch_hbm, zeros_hbm, out_hbm, buf, batchv, outv):
  c = lax.axis_index("c")
  s = lax.axis_index("s")
  wid = c * NS + s
  r = wid % 8
  cs = wid // 8
  base = r * 1248
  pltpu.sync_copy(batch_hbm.at[pl.ds(base, 1264)], batchv.at[pl.ds(0, 1264)])
  pltpu.sync_copy(zeros_hbm, outv)
  for half in range(2):
    pltpu.sync_copy(h3_hbm.at[cs, pl.ds(base + half * 632, 632)], buf)

    def row(k, _):
      seg = batchv[pl.ds(half * 632 + k, 16)][0]
      for j in range(8):
        v = buf[k, pl.ds(j * 16, 16)]
        m = outv[seg, pl.ds(j * 16, 16)]
        outv[seg, pl.ds(j * 16, 16)] = jnp.maximum(m, v)
      return 0

    lax.fori_loop(0, 632, row, 0)
  pltpu.sync_copy(outv, out_hbm.at[cs, r])


_NB = 1000


def _row_spec(width):
  return pl.BlockSpec((_NB, width), lambda i: (i, 0))


def _half_spec():
  return pl.BlockSpec((1, _NB, 128), lambda i: (i // 5, i % 5, 0))


def _full_spec(shape):
  nd = len(shape)
  return pl.BlockSpec(shape, lambda i, _n=nd: (0,) * _n)


def _tc_pre(sdeg, x):
  def body(sdeg_r, x_r, dinv_o, t_o):
    deg = 1.0 + sdeg_r[0][:, 0]
    dinv = lax.rsqrt(deg)[:, None]
    dinv_o[...] = dinv
    t_o[...] = dinv * x_r[...]

  return pl.pallas_call(
      body,
      grid=(N // _NB,),
      in_specs=[_half_spec(), _row_spec(128)],
      out_specs=[_row_spec(1), _row_spec(128)],
      out_shape=[
          jax.ShapeDtypeStruct((N, 1), _F32),
          jax.ShapeDtypeStruct((N, 128), _F32),
      ],
  )(sdeg, x)


def _tc_layer12(s_part, t, dinv, W, b):
  fo = W.shape[1]
  ng = fo // 128

  def body(s_r, t_r, dinv_r, w_r, b_r, *outs):
    di = dinv_r[...]
    agg = di * (s_r[0] + t_r[...])
    h = jnp.dot(agg, w_r[...], preferred_element_type=_F32) + b_r[...]
    tn = di * jnp.maximum(h, 0.0)
    for k in range(ng):
      outs[k][...] = tn[:, 128 * k:128 * (k + 1)]

  return pl.pallas_call(
      body,
      grid=(N // _NB,),
      in_specs=[
          _half_spec(),
          _row_spec(128),
          _row_spec(1),
          _full_spec(W.shape),
          _full_spec(b.shape),
      ],
      out_specs=[_row_spec(128)] * ng,
      out_shape=[jax.ShapeDtypeStruct((N, 128), _F32)] * ng,
  )(s_part, t, dinv, W, b)


def _tc_layer3(s01, s23, t01, t23, dinv, W, b):
  def body(s01_r, s23_r, t01_r, t23_r, dinv_r, w_r, b_r, o):
    di = dinv_r[...]
    s_full = jnp.concatenate([s01_r[0], s23_r[0]], axis=1)
    t_full = jnp.concatenate([t01_r[...], t23_r[...]], axis=1)
    agg = di * (s_full + t_full)
    h = jnp.dot(agg, w_r[...], preferred_element_type=_F32) + b_r[...]
    h = jnp.maximum(h, 0.0)
    o[...] = jnp.stack(
        [h[:, 0:128], h[:, 128:256], h[:, 256:384], h[:, 384:512]], axis=0)

  return pl.pallas_call(
      body,
      grid=(N // _NB,),
      in_specs=[
          _half_spec(),
          _half_spec(),
          _row_spec(128),
          _row_spec(128),
          _row_spec(1),
          _full_spec(W.shape),
          _full_spec(b.shape),
      ],
      out_specs=pl.BlockSpec((4, _NB, 128), lambda i: (0, i, 0)),
      out_shape=jax.ShapeDtypeStruct((4, N, 128), _F32),
  )(s01, s23, t01, t23, dinv, W, b)


def _tc_head(part, Wf1, bf1, Wf2, bf2):
  def body(p_r, w1_r, b1_r, w2_r, b2_r, o):
    p = p_r[...]
    strips = [jnp.max(p[8 * i:8 * (i + 1)], axis=0) for i in range(4)]
    pooled = jnp.concatenate(strips, axis=1)
    g = jnp.dot(pooled, w1_r[...], preferred_element_type=_F32) + b1_r[...]
    g = jnp.maximum(g, 0.0)
    o[...] = jnp.dot(g, w2_r[...], preferred_element_type=_F32) + b2_r[...]

  return pl.pallas_call(
      body,
      out_shape=jax.ShapeDtypeStruct((G, 128), _F32),
  )(part, Wf1, bf1, Wf2, bf2)


@jax.jit
def _impl(x, edge_index, batch, W1, b1, W2, b2, W3, b3, Wf1, bf1, Wf2, bf2):
  src = edge_index[0]
  dst = edge_index[1]
  srcC = src.reshape(NS, NCH, CH)
  low = dst < HN
  dst0 = jnp.where(low, dst, HN).reshape(NS, NCH, CH)
  dst1 = jnp.where(low, HN, dst - HN).reshape(NS, NCH, CH)
  zeros = jnp.zeros((G, 128), _F32)
  ones = jnp.ones((N, 128), _F32)
  idxz = jnp.zeros((NS, NCH, CH), jnp.int32)

  agg = _agg_sc()
  sdeg = agg(ones, idxz, dst0, dst1, zeros)
  dinv, t1 = _tc_pre(sdeg, x)
  s1 = agg(t1, srcC, dst0, dst1, zeros)
  (t2,) = _tc_layer12(s1, t1, dinv, W1, b1.reshape(1, -1))
  s2 = agg(t2, srcC, dst0, dst1, zeros)
  t01, t23 = _tc_layer12(s2, t2, dinv, W2, b2.reshape(1, -1))
  s01 = agg(t01, srcC, dst0, dst1, zeros)
  s23 = agg(t23, srcC, dst0, dst1, zeros)
  h3 = _tc_layer3(s01, s23, t01, t23, dinv, W3, b3.reshape(1, -1))
  part = _pool_sc()(h3, batch, zeros)
  return _tc_head(part.reshape(NC * NS, G, 128), Wf1, bf1.reshape(1, -1),
                  Wf2, bf2.reshape(1, -1))


def kernel(x, edge_index, batch, W1, b1, W2, b2, W3, b3, Wf1, bf1, Wf2, bf2):
  return _impl(x, edge_index, batch, W1, b1, W2, b2, W3, b3,
               Wf1, bf1, Wf2, bf2)


# speedup vs baseline: 15.0971x; 15.0971x over previous
import functools

import jax
import jax.numpy as jnp
from jax import lax
from jax.experimental import pallas as pl
from jax.experimental.pallas import tpu as pltpu
from jax.experimental.pallas import tpu_sc as plsc

N = 10000
E = 320000
G = 128
NC = 2
NS = 16
HN = N // NC
CH = 125
EPT = E // NS
NCH = EPT // CH
NBUF = 2
SPAN, CZ = 320, 40

_F32 = jnp.float32


@functools.cache
def _mesh():
  return plsc.VectorSubcoreMesh(core_axis_name="c", subcore_axis_name="s")


@functools.cache
def _agg_sc():
  return pl.kernel(
      _agg_sc_body,
      out_type=jax.ShapeDtypeStruct((NC, HN, 128), _F32),
      mesh=_mesh(),
      scratch_types=[
          pltpu.VMEM((NCH, CH), jnp.int32),
          pltpu.VMEM((NCH, CH), jnp.int32),
          [pltpu.VMEM((CH, 128), _F32)] * NBUF,
          pltpu.VMEM((CZ, 128), _F32),
          pltpu.VMEM_SHARED((HN + 8, 128), _F32),
          [pltpu.SemaphoreType.DMA] * NBUF,
      ])


def _agg_sc_body(t_hbm, src_hbm, dst0_hbm, dst1_hbm, zeros_hbm, out_hbm,
                 idx_s, idx_d, rows, zbuf, acc, sem):
  c = lax.axis_index("c")
  s = lax.axis_index("s")
  pltpu.sync_copy(src_hbm.at[s], idx_s)

  @pl.when(c == 0)
  def _():
    pltpu.sync_copy(dst0_hbm.at[s], idx_d)

  @pl.when(c == 1)
  def _():
    pltpu.sync_copy(dst1_hbm.at[s], idx_d)

  pltpu.sync_copy(zeros_hbm.at[pl.ds(0, CZ)], zbuf)

  def zcp(k, _):
    pltpu.sync_copy(zbuf, acc.at[pl.ds(s * SPAN + k * CZ, CZ)])
    return 0

  lax.fori_loop(0, 5, zcp, 0)

  @pl.when(s < NS - 1)
  def _():
    lax.fori_loop(5, 8, zcp, 0)

  plsc.subcore_barrier()

  def group(g, _):
    base = NBUF * g
    started = [
        pltpu.async_copy(t_hbm.at[idx_s.at[base + k]], rows[k], sem[k])
        for k in range(NBUF)
    ]
    for k in range(NBUF):
      started[k].wait()
      pltpu.sync_copy(rows[k], acc.at[idx_d.at[base + k]], add=True)
    return 0

  lax.fori_loop(0, NCH // NBUF, group, 0)
  plsc.subcore_barrier()

  def rcp(k, _):
    o = s * SPAN + k * CZ
    pltpu.sync_copy(acc.at[pl.ds(o, CZ)], out_hbm.at[c, pl.ds(o, CZ)])
    return 0

  lax.fori_loop(0, 5, rcp, 0)

  @pl.when(s < NS - 1)
  def _():
    lax.fori_loop(5, 8, rcp, 0)


@functools.cache
def _pool_sc():
  return pl.kernel(
      _pool_sc_body,
      out_type=jax.ShapeDtypeStruct((4, 8, G, 128), _F32),
      mesh=_mesh(),
      scratch_types=[
          pltpu.VMEM((632, 128), _F32),
          pltpu.VMEM((1280,), jnp.int32),
          pltpu.VMEM((G, 128), _F32),
      ])


def _pool_sc_body(h3_hbm, batch_hbm, zeros_hbm, out_hbm, buf, batchv, outv):
  c = lax.axis_index("c")
  s = lax.axis_index("s")
  wid = c * NS + s
  r = wid % 8
  cs = wid // 8
  base = r * 1248
  pltpu.sync_copy(batch_hbm.at[pl.ds(base, 1264)], batchv.at[pl.ds(0, 1264)])
  pltpu.sync_copy(zeros_hbm, outv)
  for half in range(2):
    pltpu.sync_copy(h3_hbm.at[cs, pl.ds(base + half * 632, 632)], buf)

    def row(k, _):
      seg = batchv[pl.ds(half * 632 + k, 16)][0]
      for j in range(8):
        v = buf[k, pl.ds(j * 16, 16)]
        m = outv[seg, pl.ds(j * 16, 16)]
        outv[seg, pl.ds(j * 16, 16)] = jnp.maximum(m, v)
      return 0

    lax.fori_loop(0, 632, row, 0)
  pltpu.sync_copy(outv, out_hbm.at[cs, r])


_NB = 1000


def _row_spec(width):
  return pl.BlockSpec((_NB, width), lambda i: (i, 0))


def _half_spec():
  return pl.BlockSpec((1, _NB, 128), lambda i: (i // 5, i % 5, 0))


def _full_spec(shape):
  nd = len(shape)
  return pl.BlockSpec(shape, lambda i, _n=nd: (0,) * _n)


def _tc_pre(sdeg, x):
  def body(sdeg_r, x_r, dinv_o, t_o):
    deg = 1.0 + sdeg_r[0][:, 0]
    dinv = lax.rsqrt(deg)[:, None]
    dinv_o[...] = dinv
    t_o[...] = dinv * x_r[...]

  return pl.pallas_call(
      body,
      grid=(N // _NB,),
      in_specs=[_half_spec(), _row_spec(128)],
      out_specs=[_row_spec(1), _row_spec(128)],
      out_shape=[
          jax.ShapeDtypeStruct((N, 1), _F32),
          jax.ShapeDtypeStruct((N, 128), _F32),
      ],
  )(sdeg, x)


def _tc_layer12(s_part, t, dinv, W, b):
  fo = W.shape[1]
  ng = fo // 128

  def body(s_r, t_r, dinv_r, w_r, b_r, *outs):
    di = dinv_r[...]
    agg = di * (s_r[0] + t_r[...])
    h = jnp.dot(agg, w_r[...], preferred_element_type=_F32) + b_r[...]
    tn = di * jnp.maximum(h, 0.0)
    for k in range(ng):
      outs[k][...] = tn[:, 128 * k:128 * (k + 1)]

  return pl.pallas_call(
      body,
      grid=(N // _NB,),
      in_specs=[
          _half_spec(),
          _row_spec(128),
          _row_spec(1),
          _full_spec(W.shape),
          _full_spec(b.shape),
      ],
      out_specs=[_row_spec(128)] * ng,
      out_shape=[jax.ShapeDtypeStruct((N, 128), _F32)] * ng,
  )(s_part, t, dinv, W, b)


def _tc_layer3(s01, s23, t01, t23, dinv, W, b):
  def body(s01_r, s23_r, t01_r, t23_r, dinv_r, w_r, b_r, o):
    di = dinv_r[...]
    s_full = jnp.concatenate([s01_r[0], s23_r[0]], axis=1)
    t_full = jnp.concatenate([t01_r[...], t23_r[...]], axis=1)
    agg = di * (s_full + t_full)
    h = jnp.dot(agg, w_r[...], preferred_element_type=_F32) + b_r[...]
    h = jnp.maximum(h, 0.0)
    o[...] = jnp.stack(
        [h[:, 0:128], h[:, 128:256], h[:, 256:384], h[:, 384:512]], axis=0)

  return pl.pallas_call(
      body,
      grid=(N // _NB,),
      in_specs=[
          _half_spec(),
          _half_spec(),
          _row_spec(128),
          _row_spec(128),
          _row_spec(1),
          _full_spec(W.shape),
          _full_spec(b.shape),
      ],
      out_specs=pl.BlockSpec((4, _NB, 128), lambda i: (0, i, 0)),
      out_shape=jax.ShapeDtypeStruct((4, N, 128), _F32),
  )(s01, s23, t01, t23, dinv, W, b)


def _tc_head(part, Wf1, bf1, Wf2, bf2):
  def body(p_r, w1_r, b1_r, w2_r, b2_r, o):
    p = p_r[...]
    strips = [jnp.max(p[8 * i:8 * (i + 1)], axis=0) for i in range(4)]
    pooled = jnp.concatenate(strips, axis=1)
    g = jnp.dot(pooled, w1_r[...], preferred_element_type=_F32) + b1_r[...]
    g = jnp.maximum(g, 0.0)
    o[...] = jnp.dot(g, w2_r[...], preferred_element_type=_F32) + b2_r[...]

  return pl.pallas_call(
      body,
      out_shape=jax.ShapeDtypeStruct((G, 128), _F32),
  )(part, Wf1, bf1, Wf2, bf2)


@jax.jit
def _impl(x, edge_index, batch, W1, b1, W2, b2, W3, b3, Wf1, bf1, Wf2, bf2):
  src = edge_index[0]
  dst = edge_index[1]
  srcC = src.reshape(NS, NCH, CH)
  low = dst < HN
  dst0 = jnp.where(low, dst, HN).reshape(NS, NCH, CH)
  dst1 = jnp.where(low, HN, dst - HN).reshape(NS, NCH, CH)
  zeros = jnp.zeros((G, 128), _F32)
  ones = jnp.ones((N, 128), _F32)

  agg = _agg_sc()
  sdeg = agg(ones, srcC, dst0, dst1, zeros)
  dinv, t1 = _tc_pre(sdeg, x)
  s1 = agg(t1, srcC, dst0, dst1, zeros)
  (t2,) = _tc_layer12(s1, t1, dinv, W1, b1.reshape(1, -1))
  s2 = agg(t2, srcC, dst0, dst1, zeros)
  t01, t23 = _tc_layer12(s2, t2, dinv, W2, b2.reshape(1, -1))
  s01 = agg(t01, srcC, dst0, dst1, zeros)
  s23 = agg(t23, srcC, dst0, dst1, zeros)
  h3 = _tc_layer3(s01, s23, t01, t23, dinv, W3, b3.reshape(1, -1))
  part = _pool_sc()(h3, batch, zeros)
  return _tc_head(part.reshape(NC * NS, G, 128), Wf1, bf1.reshape(1, -1),
                  Wf2, bf2.reshape(1, -1))


def kernel(x, edge_index, batch, W1, b1, W2, b2, W3, b3, Wf1, bf1, Wf2, bf2):
  return _impl(x, edge_index, batch, W1, b1, W2, b2, W3, b3,
               Wf1, bf1, Wf2, bf2)


# speedup vs baseline: 16.1582x; 1.0703x over previous
import functools

import jax
import jax.numpy as jnp
from jax import lax
from jax.experimental import pallas as pl
from jax.experimental.pallas import tpu as pltpu
from jax.experimental.pallas import tpu_sc as plsc

N = 10000
E = 320000
G = 128
NC = 2
NS = 16
HN = N // NC
CH = 125
EPT = E // NS
NCH = EPT // CH
NBUF = 2
SPAN, CZ = 320, 40

_F32 = jnp.float32


@functools.cache
def _mesh():
  return plsc.VectorSubcoreMesh(core_axis_name="c", subcore_axis_name="s")


@functools.cache
def _deg_sc():
  return pl.kernel(
      _deg_sc_body,
      out_type=jax.ShapeDtypeStruct((NC, HN, 16), _F32),
      mesh=_mesh(),
      scratch_types=[
          pltpu.VMEM((NCH, CH), jnp.int32),
          pltpu.VMEM((CH, 16), _F32),
          pltpu.VMEM((CZ, 16), _F32),
          pltpu.VMEM_SHARED((HN + 8, 16), _F32),
      ])


def _deg_sc_body(ones_hbm, dst0_hbm, dst1_hbm, zeros_hbm, out_hbm,
                 idx_d, onesbuf, zbuf, acc):
  c = lax.axis_index("c")
  s = lax.axis_index("s")

  @pl.when(c == 0)
  def _():
    pltpu.sync_copy(dst0_hbm.at[s], idx_d)

  @pl.when(c == 1)
  def _():
    pltpu.sync_copy(dst1_hbm.at[s], idx_d)

  pltpu.sync_copy(ones_hbm, onesbuf)
  pltpu.sync_copy(zeros_hbm, zbuf)

  def zcp(k, _):
    pltpu.sync_copy(zbuf, acc.at[pl.ds(s * SPAN + k * CZ, CZ)])
    return 0

  lax.fori_loop(0, 5, zcp, 0)

  @pl.when(s < NS - 1)
  def _():
    lax.fori_loop(5, 8, zcp, 0)

  plsc.subcore_barrier()

  def chunk(j, _):
    pltpu.sync_copy(onesbuf, acc.at[idx_d.at[j]], add=True)
    return 0

  lax.fori_loop(0, NCH, chunk, 0)
  plsc.subcore_barrier()

  def rcp(k, _):
    o = s * SPAN + k * CZ
    pltpu.sync_copy(acc.at[pl.ds(o, CZ)], out_hbm.at[c, pl.ds(o, CZ)])
    return 0

  lax.fori_loop(0, 5, rcp, 0)

  @pl.when(s < NS - 1)
  def _():
    lax.fori_loop(5, 8, rcp, 0)


@functools.cache
def _agg_sc():
  return pl.kernel(
      _agg_sc_body,
      out_type=jax.ShapeDtypeStruct((NC, HN, 128), _F32),
      mesh=_mesh(),
      scratch_types=[
          pltpu.VMEM((NCH, CH), jnp.int32),
          pltpu.VMEM((NCH, CH), jnp.int32),
          [pltpu.VMEM((CH, 128), _F32)] * NBUF,
          pltpu.VMEM((CZ, 128), _F32),
          pltpu.VMEM_SHARED((HN + 8, 128), _F32),
          [pltpu.SemaphoreType.DMA] * NBUF,
      ])


def _agg_sc_body(t_hbm, src_hbm, dst0_hbm, dst1_hbm, zeros_hbm, out_hbm,
                 idx_s, idx_d, rows, zbuf, acc, sem):
  c = lax.axis_index("c")
  s = lax.axis_index("s")
  pltpu.sync_copy(src_hbm.at[s], idx_s)

  @pl.when(c == 0)
  def _():
    pltpu.sync_copy(dst0_hbm.at[s], idx_d)

  @pl.when(c == 1)
  def _():
    pltpu.sync_copy(dst1_hbm.at[s], idx_d)

  pltpu.sync_copy(zeros_hbm.at[pl.ds(0, CZ)], zbuf)

  def zcp(k, _):
    pltpu.sync_copy(zbuf, acc.at[pl.ds(s * SPAN + k * CZ, CZ)])
    return 0

  lax.fori_loop(0, 5, zcp, 0)

  @pl.when(s < NS - 1)
  def _():
    lax.fori_loop(5, 8, zcp, 0)

  plsc.subcore_barrier()

  def group(g, _):
    base = NBUF * g
    started = [
        pltpu.async_copy(t_hbm.at[idx_s.at[base + k]], rows[k], sem[k])
        for k in range(NBUF)
    ]
    for k in range(NBUF):
      started[k].wait()
      pltpu.sync_copy(rows[k], acc.at[idx_d.at[base + k]], add=True)
    return 0

  lax.fori_loop(0, NCH // NBUF, group, 0)
  plsc.subcore_barrier()

  def rcp(k, _):
    o = s * SPAN + k * CZ
    pltpu.sync_copy(acc.at[pl.ds(o, CZ)], out_hbm.at[c, pl.ds(o, CZ)])
    return 0

  lax.fori_loop(0, 5, rcp, 0)

  @pl.when(s < NS - 1)
  def _():
    lax.fori_loop(5, 8, rcp, 0)


@functools.cache
def _pool_sc():
  return pl.kernel(
      _pool_sc_body,
      out_type=jax.ShapeDtypeStruct((4, 8, G, 128), _F32),
      mesh=_mesh(),
      scratch_types=[
          pltpu.VMEM((632, 128), _F32),
          pltpu.VMEM((1280,), jnp.int32),
          pltpu.VMEM((G, 128), _F32),
      ])


def _pool_sc_body(h3_hbm, batch_hbm, zeros_hbm, out_hbm, buf, batchv, outv):
  c = lax.axis_index("c")
  s = lax.axis_index("s")
  wid = c * NS + s
  r = wid % 8
  cs = wid // 8
  base = r * 1248
  pltpu.sync_copy(batch_hbm.at[pl.ds(base, 1264)], batchv.at[pl.ds(0, 1264)])
  pltpu.sync_copy(zeros_hbm, outv)
  for half in range(2):
    pltpu.sync_copy(h3_hbm.at[cs, pl.ds(base + half * 632, 632)], buf)

    def row(k, _):
      seg = batchv[pl.ds(half * 632 + k, 16)][0]
      for j in range(8):
        v = buf[k, pl.ds(j * 16, 16)]
        m = outv[seg, pl.ds(j * 16, 16)]
        outv[seg, pl.ds(j * 16, 16)] = jnp.maximum(m, v)
      return 0

    lax.fori_loop(0, 632, row, 0)
  pltpu.sync_copy(outv, out_hbm.at[cs, r])


_NB = 1000


def _row_spec(width):
  return pl.BlockSpec((_NB, width), lambda i: (i, 0))


def _half_spec():
  return pl.BlockSpec((1, _NB, 128), lambda i: (i // 5, i % 5, 0))


def _full_spec(shape):
  nd = len(shape)
  return pl.BlockSpec(shape, lambda i, _n=nd: (0,) * _n)


def _tc_pre(sdeg, x):
  def body(sdeg_r, x_r, dinv_o, t_o):
    deg = 1.0 + sdeg_r[0][:, 0]
    dinv = lax.rsqrt(deg)[:, None]
    dinv_o[...] = dinv
    t_o[...] = dinv * x_r[...]

  return pl.pallas_call(
      body,
      grid=(N // _NB,),
      in_specs=[
          pl.BlockSpec((1, _NB, 16), lambda i: (i // 5, i % 5, 0)),
          _row_spec(128),
      ],
      out_specs=[_row_spec(1), _row_spec(128)],
      out_shape=[
          jax.ShapeDtypeStruct((N, 1), _F32),
          jax.ShapeDtypeStruct((N, 128), _F32),
      ],
  )(sdeg, x)


def _tc_layer12(s_part, t, dinv, W, b):
  fo = W.shape[1]
  ng = fo // 128

  def body(s_r, t_r, dinv_r, w_r, b_r, *outs):
    di = dinv_r[...]
    agg = di * (s_r[0] + t_r[...])
    h = jnp.dot(agg, w_r[...], preferred_element_type=_F32) + b_r[...]
    tn = di * jnp.maximum(h, 0.0)
    for k in range(ng):
      outs[k][...] = tn[:, 128 * k:128 * (k + 1)]

  return pl.pallas_call(
      body,
      grid=(N // _NB,),
      in_specs=[
          _half_spec(),
          _row_spec(128),
          _row_spec(1),
          _full_spec(W.shape),
          _full_spec(b.shape),
      ],
      out_specs=[_row_spec(128)] * ng,
      out_shape=[jax.ShapeDtypeStruct((N, 128), _F32)] * ng,
  )(s_part, t, dinv, W, b)


def _tc_layer3(s01, s23, t01, t23, dinv, W, b):
  def body(s01_r, s23_r, t01_r, t23_r, dinv_r, w_r, b_r, o):
    di = dinv_r[...]
    s_full = jnp.concatenate([s01_r[0], s23_r[0]], axis=1)
    t_full = jnp.concatenate([t01_r[...], t23_r[...]], axis=1)
    agg = di * (s_full + t_full)
    h = jnp.dot(agg, w_r[...], preferred_element_type=_F32) + b_r[...]
    h = jnp.maximum(h, 0.0)
    o[...] = jnp.stack(
        [h[:, 0:128], h[:, 128:256], h[:, 256:384], h[:, 384:512]], axis=0)

  return pl.pallas_call(
      body,
      grid=(N // _NB,),
      in_specs=[
          _half_spec(),
          _half_spec(),
          _row_spec(128),
          _row_spec(128),
          _row_spec(1),
          _full_spec(W.shape),
          _full_spec(b.shape),
      ],
      out_specs=pl.BlockSpec((4, _NB, 128), lambda i: (0, i, 0)),
      out_shape=jax.ShapeDtypeStruct((4, N, 128), _F32),
  )(s01, s23, t01, t23, dinv, W, b)


def _tc_head(part, Wf1, bf1, Wf2, bf2):
  def body(p_r, w1_r, b1_r, w2_r, b2_r, o):
    p = p_r[...]
    strips = [jnp.max(p[8 * i:8 * (i + 1)], axis=0) for i in range(4)]
    pooled = jnp.concatenate(strips, axis=1)
    g = jnp.dot(pooled, w1_r[...], preferred_element_type=_F32) + b1_r[...]
    g = jnp.maximum(g, 0.0)
    o[...] = jnp.dot(g, w2_r[...], preferred_element_type=_F32) + b2_r[...]

  return pl.pallas_call(
      body,
      out_shape=jax.ShapeDtypeStruct((G, 128), _F32),
  )(part, Wf1, bf1, Wf2, bf2)


@jax.jit
def _impl(x, edge_index, batch, W1, b1, W2, b2, W3, b3, Wf1, bf1, Wf2, bf2):
  src = edge_index[0]
  dst = edge_index[1]
  srcC = src.reshape(NS, NCH, CH)
  low = dst < HN
  dst0 = jnp.where(low, dst, HN).reshape(NS, NCH, CH)
  dst1 = jnp.where(low, HN, dst - HN).reshape(NS, NCH, CH)
  zeros = jnp.zeros((G, 128), _F32)
  ones16 = jnp.ones((CH, 16), _F32)
  zeros16 = jnp.zeros((CZ, 16), _F32)

  agg = _agg_sc()
  sdeg = _deg_sc()(ones16, dst0, dst1, zeros16)
  dinv, t1 = _tc_pre(sdeg, x)
  s1 = agg(t1, srcC, dst0, dst1, zeros)
  (t2,) = _tc_layer12(s1, t1, dinv, W1, b1.reshape(1, -1))
  s2 = agg(t2, srcC, dst0, dst1, zeros)
  t01, t23 = _tc_layer12(s2, t2, dinv, W2, b2.reshape(1, -1))
  s01 = agg(t01, srcC, dst0, dst1, zeros)
  s23 = agg(t23, srcC, dst0, dst1, zeros)
  h3 = _tc_layer3(s01, s23, t01, t23, dinv, W3, b3.reshape(1, -1))
  part = _pool_sc()(h3, batch, zeros)
  return _tc_head(part.reshape(NC * NS, G, 128), Wf1, bf1.reshape(1, -1),
                  Wf2, bf2.reshape(1, -1))


def kernel(x, edge_index, batch, W1, b1, W2, b2, W3, b3, Wf1, bf1, Wf2, bf2):
  return _impl(x, edge_index, batch, W1, b1, W2, b2, W3, b3,
               Wf1, bf1, Wf2, bf2)


# speedup vs baseline: 18.0572x; 1.1175x over previous
import functools

import jax
import jax.numpy as jnp
from jax import lax
from jax.experimental import pallas as pl
from jax.experimental.pallas import tpu as pltpu
from jax.experimental.pallas import tpu_sc as plsc

N = 10000
E = 320000
G = 128
NC = 2
NS = 16
HN = N // NC
CH = 125
EPT = E // NS
NCH = EPT // CH
NBUF = 2
SPAN, CZ = 320, 40

_F32 = jnp.float32


@functools.cache
def _mesh():
  return plsc.VectorSubcoreMesh(core_axis_name="c", subcore_axis_name="s")


@functools.cache
def _agg_sc():
  return pl.kernel(
      _agg_sc_body,
      out_type=jax.ShapeDtypeStruct((NC, HN, 128), _F32),
      mesh=_mesh(),
      scratch_types=[
          pltpu.VMEM((NCH, CH), jnp.int32),
          pltpu.VMEM((NCH, CH), jnp.int32),
          [pltpu.VMEM((CH, 128), _F32)] * NBUF,
          pltpu.VMEM((CZ, 128), _F32),
          pltpu.VMEM_SHARED((HN + 8, 128), _F32),
          [pltpu.SemaphoreType.DMA] * NBUF,
      ])


def _agg_sc_body(t_hbm, src_hbm, dst0_hbm, dst1_hbm, zeros_hbm, out_hbm,
                 idx_s, idx_d, rows, zbuf, acc, sem):
  c = lax.axis_index("c")
  s = lax.axis_index("s")
  pltpu.sync_copy(src_hbm.at[s], idx_s)

  @pl.when(c == 0)
  def _():
    pltpu.sync_copy(dst0_hbm.at[s], idx_d)

  @pl.when(c == 1)
  def _():
    pltpu.sync_copy(dst1_hbm.at[s], idx_d)

  pltpu.sync_copy(zeros_hbm.at[pl.ds(0, CZ)], zbuf)

  def zcp(k, _):
    pltpu.sync_copy(zbuf, acc.at[pl.ds(s * SPAN + k * CZ, CZ)])
    return 0

  lax.fori_loop(0, 5, zcp, 0)

  @pl.when(s < NS - 1)
  def _():
    lax.fori_loop(5, 8, zcp, 0)

  plsc.subcore_barrier()

  for b in range(NBUF):
    pltpu.async_copy(t_hbm.at[idx_s.at[b]], rows[b], sem[b])

  def group(g, _):
    base = NBUF * g
    for b in range(NBUF):
      j = base + b
      pltpu.make_async_copy(t_hbm.at[idx_s.at[0]], rows[b], sem[b]).wait()
      pltpu.sync_copy(rows[b], acc.at[idx_d.at[j]], add=True)

      @pl.when(g < NCH // NBUF - 1)
      def _():
        pltpu.async_copy(t_hbm.at[idx_s.at[j + NBUF]], rows[b], sem[b])

    return 0

  lax.fori_loop(0, NCH // NBUF, group, 0)
  plsc.subcore_barrier()

  def rcp(k, _):
    o = s * SPAN + k * CZ
    pltpu.sync_copy(acc.at[pl.ds(o, CZ)], out_hbm.at[c, pl.ds(o, CZ)])
    return 0

  lax.fori_loop(0, 5, rcp, 0)

  @pl.when(s < NS - 1)
  def _():
    lax.fori_loop(5, 8, rcp, 0)


@functools.cache
def _pool_sc():
  return pl.kernel(
      _pool_sc_body,
      out_type=jax.ShapeDtypeStruct((4, 8, G, 128), _F32),
      mesh=_mesh(),
      scratch_types=[
          pltpu.VMEM((632, 128), _F32),
          pltpu.VMEM((1280,), jnp.int32),
          pltpu.VMEM((G, 128), _F32),
      ])


def _pool_sc_body(h3_hbm, batch_hbm, zeros_hbm, out_hbm, buf, batchv, outv):
  c = lax.axis_index("c")
  s = lax.axis_index("s")
  wid = c * NS + s
  r = wid % 8
  cs = wid // 8
  base = r * 1248
  pltpu.sync_copy(batch_hbm.at[pl.ds(base, 1264)], batchv.at[pl.ds(0, 1264)])
  pltpu.sync_copy(zeros_hbm, outv)
  for half in range(2):
    pltpu.sync_copy(h3_hbm.at[cs, pl.ds(base + half * 632, 632)], buf)

    def row(k, _):
      seg = batchv[pl.ds(half * 632 + k, 16)][0]
      for j in range(8):
        v = buf[k, pl.ds(j * 16, 16)]
        m = outv[seg, pl.ds(j * 16, 16)]
        outv[seg, pl.ds(j * 16, 16)] = jnp.maximum(m, v)
      return 0

    lax.fori_loop(0, 632, row, 0)
  pltpu.sync_copy(outv, out_hbm.at[cs, r])


_NB = 1000


def _row_spec(width):
  return pl.BlockSpec((_NB, width), lambda i: (i, 0))


def _half_spec():
  return pl.BlockSpec((1, _NB, 128), lambda i: (i // 5, i % 5, 0))


def _full_spec(shape):
  nd = len(shape)
  return pl.BlockSpec(shape, lambda i, _n=nd: (0,) * _n)


def _tc_pre(sdeg, x):
  def body(sdeg_r, x_r, dinv_o, t_o):
    deg = 1.0 + sdeg_r[0][:, 0]
    dinv = lax.rsqrt(deg)[:, None]
    dinv_o[...] = dinv
    t_o[...] = dinv * x_r[...]

  return pl.pallas_call(
      body,
      grid=(N // _NB,),
      in_specs=[_half_spec(), _row_spec(128)],
      out_specs=[_row_spec(1), _row_spec(128)],
      out_shape=[
          jax.ShapeDtypeStruct((N, 1), _F32),
          jax.ShapeDtypeStruct((N, 128), _F32),
      ],
  )(sdeg, x)


def _tc_layer12(s_part, t, dinv, W, b):
  fo = W.shape[1]
  ng = fo // 128

  def body(s_r, t_r, dinv_r, w_r, b_r, *outs):
    di = dinv_r[...]
    agg = di * (s_r[0] + t_r[...])
    h = jnp.dot(agg, w_r[...], preferred_element_type=_F32) + b_r[...]
    tn = di * jnp.maximum(h, 0.0)
    for k in range(ng):
      outs[k][...] = tn[:, 128 * k:128 * (k + 1)]

  return pl.pallas_call(
      body,
      grid=(N // _NB,),
      in_specs=[
          _half_spec(),
          _row_spec(128),
          _row_spec(1),
          _full_spec(W.shape),
          _full_spec(b.shape),
      ],
      out_specs=[_row_spec(128)] * ng,
      out_shape=[jax.ShapeDtypeStruct((N, 128), _F32)] * ng,
  )(s_part, t, dinv, W, b)


def _tc_layer3(s01, s23, t01, t23, dinv, W, b):
  def body(s01_r, s23_r, t01_r, t23_r, dinv_r, w_r, b_r, o):
    di = dinv_r[...]
    s_full = jnp.concatenate([s01_r[0], s23_r[0]], axis=1)
    t_full = jnp.concatenate([t01_r[...], t23_r[...]], axis=1)
    agg = di * (s_full + t_full)
    h = jnp.dot(agg, w_r[...], preferred_element_type=_F32) + b_r[...]
    h = jnp.maximum(h, 0.0)
    o[...] = jnp.stack(
        [h[:, 0:128], h[:, 128:256], h[:, 256:384], h[:, 384:512]], axis=0)

  return pl.pallas_call(
      body,
      grid=(N // _NB,),
      in_specs=[
          _half_spec(),
          _half_spec(),
          _row_spec(128),
          _row_spec(128),
          _row_spec(1),
          _full_spec(W.shape),
          _full_spec(b.shape),
      ],
      out_specs=pl.BlockSpec((4, _NB, 128), lambda i: (0, i, 0)),
      out_shape=jax.ShapeDtypeStruct((4, N, 128), _F32),
  )(s01, s23, t01, t23, dinv, W, b)


def _tc_head(part, Wf1, bf1, Wf2, bf2):
  def body(p_r, w1_r, b1_r, w2_r, b2_r, o):
    p = p_r[...]
    strips = [jnp.max(p[8 * i:8 * (i + 1)], axis=0) for i in range(4)]
    pooled = jnp.concatenate(strips, axis=1)
    g = jnp.dot(pooled, w1_r[...], preferred_element_type=_F32) + b1_r[...]
    g = jnp.maximum(g, 0.0)
    o[...] = jnp.dot(g, w2_r[...], preferred_element_type=_F32) + b2_r[...]

  return pl.pallas_call(
      body,
      out_shape=jax.ShapeDtypeStruct((G, 128), _F32),
  )(part, Wf1, bf1, Wf2, bf2)


@jax.jit
def _impl(x, edge_index, batch, W1, b1, W2, b2, W3, b3, Wf1, bf1, Wf2, bf2):
  src = edge_index[0]
  dst = edge_index[1]
  srcC = src.reshape(NS, NCH, CH)
  low = dst < HN
  dst0 = jnp.where(low, dst, HN).reshape(NS, NCH, CH)
  dst1 = jnp.where(low, HN, dst - HN).reshape(NS, NCH, CH)
  zeros = jnp.zeros((G, 128), _F32)
  ones = jnp.ones((N, 128), _F32)

  agg = _agg_sc()
  sdeg = agg(ones, srcC, dst0, dst1, zeros)
  dinv, t1 = _tc_pre(sdeg, x)
  s1 = agg(t1, srcC, dst0, dst1, zeros)
  (t2,) = _tc_layer12(s1, t1, dinv, W1, b1.reshape(1, -1))
  s2 = agg(t2, srcC, dst0, dst1, zeros)
  t01, t23 = _tc_layer12(s2, t2, dinv, W2, b2.reshape(1, -1))
  s01 = agg(t01, srcC, dst0, dst1, zeros)
  s23 = agg(t23, srcC, dst0, dst1, zeros)
  h3 = _tc_layer3(s01, s23, t01, t23, dinv, W3, b3.reshape(1, -1))
  part = _pool_sc()(h3, batch, zeros)
  return _tc_head(part.reshape(NC * NS, G, 128), Wf1, bf1.reshape(1, -1),
                  Wf2, bf2.reshape(1, -1))


def kernel(x, edge_index, batch, W1, b1, W2, b2, W3, b3, Wf1, bf1, Wf2, bf2):
  return _impl(x, edge_index, batch, W1, b1, W2, b2, W3, b3,
               Wf1, bf1, Wf2, bf2)
